# Initial kernel scaffold; baseline (speedup 1.0000x reference)
import jax
import jax.numpy as jnp
from jax.experimental import pallas as pl


def kernel(keys, indices, labels, buffer, mem_indices, mem_labels, ptr, num_updates):
    raise NotImplementedError("write your pallas kernel here")


# speedup vs baseline: 1.3002x; 1.3002x over previous
import jax
import jax.numpy as jnp
from jax.experimental import pallas as pl
from jax.experimental.pallas import tpu as pltpu


def _body(ptr_sm, keys_ref, idx_ref, lab_ref, buf_ref, midx_ref, mlab_ref,
          outb_ref, outi_ref, outl_ref):
    j = pl.program_id(0)
    cb = outb_ref.shape[1]
    k_total = pl.num_programs(0) * cb
    b = keys_ref.shape[0]
    p = ptr_sm[0]
    p = jnp.clip(p, 0, k_total - b)
    slab = p // cb

    @pl.when(j == slab)
    def _():
        outb_ref[...] = keys_ref[...].T
        outi_ref[...] = idx_ref[...]
        outl_ref[...] = lab_ref[...]

    @pl.when(j != slab)
    def _():
        outb_ref[...] = buf_ref[...]
        outi_ref[...] = midx_ref[...]
        outl_ref[...] = mlab_ref[...]


def kernel(keys, indices, labels, buffer, mem_indices, mem_labels, ptr,
           num_updates):
    f, K = buffer.shape
    B = keys.shape[0]
    CB = B
    grid = (K // CB,)

    grid_spec = pltpu.PrefetchScalarGridSpec(
        num_scalar_prefetch=1,
        grid=grid,
        in_specs=[
            pl.BlockSpec((B, f), lambda j, p: (0, 0)),
            pl.BlockSpec((B,), lambda j, p: (0,)),
            pl.BlockSpec((B,), lambda j, p: (0,)),
            pl.BlockSpec((f, CB), lambda j, p: (0, j)),
            pl.BlockSpec((CB,), lambda j, p: (j,)),
            pl.BlockSpec((CB,), lambda j, p: (j,)),
        ],
        out_specs=[
            pl.BlockSpec((f, CB), lambda j, p: (0, j)),
            pl.BlockSpec((CB,), lambda j, p: (j,)),
            pl.BlockSpec((CB,), lambda j, p: (j,)),
        ],
    )

    new_buffer, new_indices, new_labels = pl.pallas_call(
        _body,
        grid_spec=grid_spec,
        out_shape=[
            jax.ShapeDtypeStruct((f, K), buffer.dtype),
            jax.ShapeDtypeStruct((K,), mem_indices.dtype),
            jax.ShapeDtypeStruct((K,), mem_labels.dtype),
        ],
    )(ptr, keys, indices, labels, buffer, mem_indices, mem_labels)

    p = ptr[0]
    is_reliable = (p + B) >= K
    new_ptr = jnp.reshape(((p + B) % K).astype(ptr.dtype), (1,))
    new_num_updates = num_updates + 1
    return (new_buffer, new_indices, new_labels, new_ptr, new_num_updates,
            is_reliable)
